# Initial kernel scaffold; baseline (speedup 1.0000x reference)
import functools

import jax
import jax.numpy as jnp
from jax import lax
from jax.experimental import pallas as pl
from jax.experimental.pallas import tpu as pltpu
from jax.experimental.pallas import tpu_sc as plsc

N = 10000
D = 128
H = 128
C = 40
CP = 48
E = 320000
NC = 2
NS = 16
L = 16
CHUNK = 128
CH_PER_SUB = 157
E_PAD = NS * CH_PER_SUB * CHUNK
RPS = N // NS
BN = 1000
GRID = N // BN

_f32 = jnp.float32


def _mm4_body(x10, x11, x20, x21, w1a, w1b, out):
    out[0] = jnp.dot(x10[...], w1a[...], preferred_element_type=_f32)
    out[1] = jnp.dot(x11[...], w1b[...], preferred_element_type=_f32)
    out[2] = jnp.dot(x20[...], w1a[...], preferred_element_type=_f32)
    out[3] = jnp.dot(x21[...], w1b[...], preferred_element_type=_f32)


def _mm4(x10, x11, x20, x21, w1a, w1b):
    xspec = pl.BlockSpec((BN, D), lambda r: (r, 0))
    wspec = pl.BlockSpec((D, H), lambda r: (0, 0))
    return pl.pallas_call(
        _mm4_body,
        grid=(GRID,),
        in_specs=[xspec, xspec, xspec, xspec, wspec, wspec],
        out_specs=pl.BlockSpec((4, BN, H), lambda r: (0, r, 0)),
        out_shape=jax.ShapeDtypeStruct((4, N, H), _f32),
    )(x10, x11, x20, x21, w1a, w1b)


def _spmm256_body(zflat, srcs, dsts, ews, zrows, out,
                  idx_s, idx_d, ewt, rows, acc, gsem):
    c = lax.axis_index("c")
    s = lax.axis_index("s")
    rbase = s * RPS
    for p in range(2):
        pltpu.sync_copy(zrows.at[pl.ds(rbase, RPS)], acc.at[pl.ds(rbase, RPS)])
        pltpu.sync_copy(srcs.at[p, s], idx_s)
        pltpu.sync_copy(dsts.at[p, s], idx_d)
        pltpu.sync_copy(ews.at[p, s], ewt)
        off = (2 * p) * N + c * N

        @pl.loop(0, CH_PER_SUB)
        def _offset(j):
            for g in range(8):
                idx_s[j, pl.ds(g * L, L)] = idx_s[j, pl.ds(g * L, L)] + off

        plsc.subcore_barrier()

        @pl.loop(0, CH_PER_SUB)
        def _edges(j):
            pltpu.async_copy(zflat.at[idx_s.at[j]], rows, gsem).wait()

            @pl.loop(0, CHUNK)
            def _scale(e):
                w = ewt[j, e]
                for g in range(8):
                    rows[e, pl.ds(g * L, L)] = rows[e, pl.ds(g * L, L)] * w

            pltpu.sync_copy(rows, acc.at[idx_d.at[j]], add=True)

        plsc.subcore_barrier()
        pltpu.sync_copy(acc.at[pl.ds(rbase, RPS)],
                        out.at[p, c, pl.ds(rbase, RPS)])
        if p == 0:
            plsc.subcore_barrier()


def _spmm256(zflat, srcs, dsts, ews, zrows):
    mesh = plsc.VectorSubcoreMesh(core_axis_name="c", subcore_axis_name="s",
                                  num_cores=NC, num_subcores=NS)
    return pl.kernel(
        _spmm256_body,
        out_type=jax.ShapeDtypeStruct((2, NC, N, H), _f32),
        mesh=mesh,
        scratch_types=[
            pltpu.VMEM((CH_PER_SUB, CHUNK), jnp.int32),
            pltpu.VMEM((CH_PER_SUB, CHUNK), jnp.int32),
            pltpu.VMEM((CH_PER_SUB, CHUNK), _f32),
            pltpu.VMEM((CHUNK, H), _f32),
            pltpu.VMEM_SHARED((N, H), _f32),
            pltpu.SemaphoreType.DMA,
        ],
    )(zflat, srcs, dsts, ews, zrows)


def _mid_body(svals, sl, b1a, b1b, f11a, f11b, f12a, f12b,
              fb11, fb12, w2t, w2b, yout):
    h1a = jnp.maximum(svals[0, 0] + b1a[...], 0.0)
    h1b = jnp.maximum(svals[0, 1] + b1b[...], 0.0)
    h2a = jnp.maximum(svals[1, 0] + b1a[...], 0.0)
    h2b = jnp.maximum(svals[1, 1] + b1b[...], 0.0)
    slv = sl[...]
    t1a = jnp.maximum(slv[0, 0] + b1a[...], 0.0)
    t1b = jnp.maximum(slv[0, 1] + b1b[...], 0.0)
    t2a = jnp.maximum(slv[1, 0] + b1a[...], 0.0)
    t2b = jnp.maximum(slv[1, 1] + b1b[...], 0.0)
    l1 = jax.nn.sigmoid(jnp.sum(t1a * f11a[...]) + jnp.sum(t1b * f11b[...])
                        + fb11[0, 0])
    l2 = jax.nn.sigmoid(jnp.sum(t2a * f12a[...]) + jnp.sum(t2b * f12b[...])
                        + fb12[0, 0])
    den = jnp.maximum(l1 + l2, 1e-12)
    w1 = l1 / den
    w2 = l2 / den
    xa = w1 * h1a + w2 * h2a
    xb = w1 * h1b + w2 * h2b
    yout[...] = (jnp.dot(xa, w2t[...], preferred_element_type=_f32)
                 + jnp.dot(xb, w2b[...], preferred_element_type=_f32))


def _mid(svals, b1a, b1b, fw11, fb11, fw12, fb12, w2pad):
    sl = svals[:, :, N - 1:N, :]
    f11a = fw11[:H].reshape(1, H)
    f11b = fw11[H:].reshape(1, H)
    f12a = fw12[:H].reshape(1, H)
    f12b = fw12[H:].reshape(1, H)
    vspec = pl.BlockSpec((1, H), lambda r: (0, 0))
    scspec = pl.BlockSpec((1, 1), lambda r: (0, 0))
    return pl.pallas_call(
        _mid_body,
        grid=(GRID,),
        in_specs=[
            pl.BlockSpec((2, 2, BN, H), lambda r: (0, 0, r, 0)),
            pl.BlockSpec((2, 2, 1, H), lambda r: (0, 0, 0, 0)),
            vspec, vspec, vspec, vspec, vspec, vspec,
            scspec, scspec,
            pl.BlockSpec((H, CP), lambda r: (0, 0)),
            pl.BlockSpec((H, CP), lambda r: (0, 0)),
        ],
        out_specs=pl.BlockSpec((BN, CP), lambda r: (r, 0)),
        out_shape=jax.ShapeDtypeStruct((N, CP), _f32),
    )(svals, sl,
      b1a.reshape(1, H), b1b.reshape(1, H), f11a, f11b, f12a, f12b,
      fb11.reshape(1, 1), fb12.reshape(1, 1), w2pad[:H], w2pad[H:])


def _spmm48_body(y, srcs, dsts, ews, zrows, out,
                 idx_s, idx_d, ewt, rows, acc, gsem):
    c = lax.axis_index("c")
    s = lax.axis_index("s")
    rbase = s * RPS
    pltpu.sync_copy(zrows.at[pl.ds(rbase, RPS)], acc.at[pl.ds(rbase, RPS)])
    pltpu.sync_copy(srcs.at[c, s], idx_s)
    pltpu.sync_copy(dsts.at[c, s], idx_d)
    pltpu.sync_copy(ews.at[c, s], ewt)
    plsc.subcore_barrier()

    @pl.loop(0, CH_PER_SUB)
    def _edges(j):
        pltpu.async_copy(y.at[idx_s.at[j]], rows, gsem).wait()

        @pl.loop(0, CHUNK)
        def _scale(e):
            w = ewt[j, e]
            for g in range(3):
                rows[e, pl.ds(g * L, L)] = rows[e, pl.ds(g * L, L)] * w

        pltpu.sync_copy(rows, acc.at[idx_d.at[j]], add=True)

    plsc.subcore_barrier()
    pltpu.sync_copy(acc.at[pl.ds(rbase, RPS)], out.at[c, pl.ds(rbase, RPS)])


def _spmm48(y, srcs, dsts, ews, zrows):
    mesh = plsc.VectorSubcoreMesh(core_axis_name="c", subcore_axis_name="s",
                                  num_cores=NC, num_subcores=NS)
    return pl.kernel(
        _spmm48_body,
        out_type=jax.ShapeDtypeStruct((NC, N, CP), _f32),
        mesh=mesh,
        scratch_types=[
            pltpu.VMEM((CH_PER_SUB, CHUNK), jnp.int32),
            pltpu.VMEM((CH_PER_SUB, CHUNK), jnp.int32),
            pltpu.VMEM((CH_PER_SUB, CHUNK), _f32),
            pltpu.VMEM((CHUNK, CP), _f32),
            pltpu.VMEM_SHARED((N, CP), _f32),
            pltpu.SemaphoreType.DMA,
        ],
    )(y, srcs, dsts, ews, zrows)


def _fin_body(tvals, tl, b2p, f1, f2, fb1, fb2, out, gp1o, gp2o):
    gp1 = tvals[0] + b2p[...]
    gp2 = tvals[1] + b2p[...]
    tlv = tl[...]
    g1l = tlv[0] + b2p[...]
    g2l = tlv[1] + b2p[...]
    l1 = jax.nn.sigmoid(jnp.sum(g1l * f1[...]) + fb1[0, 0])
    l2 = jax.nn.sigmoid(jnp.sum(g2l * f2[...]) + fb2[0, 0])
    den = jnp.maximum(l1 + l2, 1e-12)
    out[...] = (l1 / den) * gp1 + (l2 / den) * gp2
    gp1o[...] = gp1
    gp2o[...] = gp2


def _fin(tvals, b2p, fw1p, fb1, fw2p, fb2):
    tl = tvals[:, N - 1, :]
    vspec = pl.BlockSpec((1, CP), lambda r: (0, 0))
    scspec = pl.BlockSpec((1, 1), lambda r: (0, 0))
    oshape = jax.ShapeDtypeStruct((N, CP), _f32)
    ospec = pl.BlockSpec((BN, CP), lambda r: (r, 0))
    return pl.pallas_call(
        _fin_body,
        grid=(GRID,),
        in_specs=[pl.BlockSpec((2, BN, CP), lambda r: (0, r, 0)),
                  pl.BlockSpec((2, CP), lambda r: (0, 0)),
                  vspec, vspec, vspec, scspec, scspec],
        out_specs=[ospec, ospec, ospec],
        out_shape=[oshape, oshape, oshape],
    )(tvals, tl, b2p.reshape(1, CP), fw1p.reshape(1, CP),
      fw2p.reshape(1, CP), fb1.reshape(1, 1), fb2.reshape(1, 1))


def _pad_edges(ei, ew):
    dst = ei[0].astype(jnp.int32)
    src = ei[1].astype(jnp.int32)
    pad = E_PAD - E
    dst = jnp.concatenate([dst, jnp.zeros((pad,), jnp.int32)])
    src = jnp.concatenate([src, jnp.zeros((pad,), jnp.int32)])
    ewp = jnp.concatenate([ew.astype(_f32), jnp.zeros((pad,), _f32)])
    shape = (NS, CH_PER_SUB, CHUNK)
    return dst.reshape(shape), src.reshape(shape), ewp.reshape(shape)


def kernel(x1_0, x1_1, x2_0, x2_1, edge_index1, edge_weight1,
           edge_index2, edge_weight2, W1a, b1a, W1b, b1b, W2, b2,
           fw11, fb11, fw12, fb12, fw1, fb1, fw2, fb2):
    d1, s1, w1 = _pad_edges(edge_index1, edge_weight1)
    d2, s2, w2 = _pad_edges(edge_index2, edge_weight2)
    srcs = jnp.stack([s1, s2])
    dsts = jnp.stack([d1, d2])
    ews = jnp.stack([w1, w2])

    z = _mm4(x1_0, x1_1, x2_0, x2_1, W1a, W1b)
    zflat = z.reshape(4 * N, H)

    zrows_h = jnp.zeros((N, H), _f32)
    svals = _spmm256(zflat, srcs, dsts, ews, zrows_h)

    w2pad = jnp.concatenate([W2, jnp.zeros((2 * H, CP - C), _f32)], axis=1)
    y = _mid(svals, b1a, b1b, fw11, fb11, fw12, fb12, w2pad)

    zrows_c = jnp.zeros((N, CP), _f32)
    tvals = _spmm48(y, srcs, dsts, ews, zrows_c)

    b2p = jnp.concatenate([b2, jnp.zeros((CP - C,), _f32)])
    fw1p = jnp.concatenate([fw1[:, 0], jnp.zeros((CP - C,), _f32)])
    fw2p = jnp.concatenate([fw2[:, 0], jnp.zeros((CP - C,), _f32)])
    out, gp1, gp2 = _fin(tvals, b2p, fw1p, fb1, fw2p, fb2)
    return (out[:, :C], gp1[:, :C], gp2[:, :C])


# speedup vs baseline: 3.9566x; 3.9566x over previous
import jax
import jax.numpy as jnp
from jax import lax
from jax.experimental import pallas as pl
from jax.experimental.pallas import tpu as pltpu
from jax.experimental.pallas import tpu_sc as plsc

N = 10000
D = 128
H = 128
C = 40
CP = 48
FQ = 64
E = 320000
NC = 2
NS = 16
L = 16
CHUNK = 128
CH_PER_SUB = 157
E_PAD = NS * CH_PER_SUB * CHUNK
NP = 10240
RPS = NP // NS
BN = 1000
GRID = N // BN

_f32 = jnp.float32

_GDN = lax.GatherDimensionNumbers(offset_dims=(), collapsed_slice_dims=(0,),
                                  start_index_map=(0,))


def _lane_splat(vec, lane):
    idx = jnp.full((L, 1), lane, jnp.int32)
    return lax.gather(vec, idx, _GDN, (1,),
                      mode=lax.GatherScatterMode.PROMISE_IN_BOUNDS)


def _mm4_body(x10, x11, x20, x21, w1a, w1b, out):
    z = [jnp.dot(x10[...], w1a[...], preferred_element_type=_f32),
         jnp.dot(x11[...], w1b[...], preferred_element_type=_f32),
         jnp.dot(x20[...], w1a[...], preferred_element_type=_f32),
         jnp.dot(x21[...], w1b[...], preferred_element_type=_f32)]
    for p in range(2):
        for ab in range(2):
            zi = z[2 * p + ab]
            out[p, 2 * ab] = zi[:, :FQ]
            out[p, 2 * ab + 1] = zi[:, FQ:]


def _mm4(x10, x11, x20, x21, w1a, w1b):
    xspec = pl.BlockSpec((BN, D), lambda r: (r, 0))
    wspec = pl.BlockSpec((D, H), lambda r: (0, 0))
    return pl.pallas_call(
        _mm4_body,
        grid=(GRID,),
        in_specs=[xspec, xspec, xspec, xspec, wspec, wspec],
        out_specs=pl.BlockSpec((2, 4, BN, FQ), lambda r: (0, 0, r, 0)),
        out_shape=jax.ShapeDtypeStruct((2, 4, N, FQ), _f32),
    )(x10, x11, x20, x21, w1a, w1b)


def _spmm256_body(zflat, srcs, dsts, ews, zrows, out,
                  idx_s, idx_d, ewt, sidx, didx, ewc, rows, acc, gsem):
    c = lax.axis_index("c")
    s = lax.axis_index("s")
    rbase = s * RPS
    for p in range(2):
        pltpu.sync_copy(srcs.at[p, s], idx_s)
        pltpu.sync_copy(dsts.at[p, s], idx_d)
        pltpu.sync_copy(ews.at[p, s], ewt)
        for h in range(2):
            pltpu.sync_copy(zrows.at[pl.ds(rbase, RPS)],
                            acc.at[pl.ds(rbase, RPS)])
            off = (p * 4 + h) * N + c * (2 * N)
            plsc.subcore_barrier()

            @pl.loop(0, CH_PER_SUB)
            def _edges(j):
                for g in range(CHUNK // L):
                    sidx[pl.ds(g * L, L)] = idx_s[j, pl.ds(g * L, L)] + off
                    didx[pl.ds(g * L, L)] = idx_d[j, pl.ds(g * L, L)]
                    ewc[pl.ds(g * L, L)] = ewt[j, pl.ds(g * L, L)]
                pltpu.async_copy(zflat.at[sidx], rows, gsem).wait()

                @pl.loop(0, CHUNK // L)
                def _grp(k):
                    wg = ewc[pl.ds(k * L, L)]
                    for u in range(L):
                        e = k * L + u
                        wv = _lane_splat(wg, u)
                        for g in range(FQ // L):
                            rows[e, pl.ds(g * L, L)] = (
                                rows[e, pl.ds(g * L, L)] * wv)

                pltpu.sync_copy(rows, acc.at[didx], add=True)

            plsc.subcore_barrier()
            pltpu.sync_copy(acc.at[pl.ds(rbase, RPS)],
                            out.at[p, c, h, pl.ds(rbase, RPS)])
            plsc.subcore_barrier()


def _spmm256(zflat, srcs, dsts, ews, zrows):
    mesh = plsc.VectorSubcoreMesh(core_axis_name="c", subcore_axis_name="s",
                                  num_cores=NC, num_subcores=NS)
    return pl.kernel(
        _spmm256_body,
        out_type=jax.ShapeDtypeStruct((2, NC, 2, NP, FQ), _f32),
        mesh=mesh,
        compiler_params=pltpu.CompilerParams(use_tc_tiling_on_sc=False),
        scratch_types=[
            pltpu.VMEM((CH_PER_SUB, CHUNK), jnp.int32),
            pltpu.VMEM((CH_PER_SUB, CHUNK), jnp.int32),
            pltpu.VMEM((CH_PER_SUB, CHUNK), _f32),
            pltpu.VMEM((CHUNK,), jnp.int32),
            pltpu.VMEM((CHUNK,), jnp.int32),
            pltpu.VMEM((CHUNK,), _f32),
            pltpu.VMEM((CHUNK, FQ), _f32),
            pltpu.VMEM_SHARED((NP, FQ), _f32),
            pltpu.SemaphoreType.DMA,
        ],
    )(zflat, srcs, dsts, ews, zrows)


def _mid_body(svals, sl, b1, f11, f12, fb11, fb12, w2q, yout):
    sv = svals[...]
    slv = sl[...]
    b1v = b1[...]
    f11v = f11[...]
    f12v = f12[...]
    hq = {}
    a1 = jnp.zeros((), _f32)
    a2 = jnp.zeros((), _f32)
    for c in range(2):
        for h in range(2):
            q = 2 * c + h
            b = b1v[q]
            hq[(0, q)] = jnp.maximum(sv[0, c, h] + b, 0.0)
            hq[(1, q)] = jnp.maximum(sv[1, c, h] + b, 0.0)
            t1 = jnp.maximum(slv[0, c, h] + b, 0.0)
            t2 = jnp.maximum(slv[1, c, h] + b, 0.0)
            a1 = a1 + jnp.sum(t1 * f11v[q])
            a2 = a2 + jnp.sum(t2 * f12v[q])
    l1 = jax.nn.sigmoid(a1 + fb11[0, 0])
    l2 = jax.nn.sigmoid(a2 + fb12[0, 0])
    den = jnp.maximum(l1 + l2, 1e-12)
    w1 = l1 / den
    w2 = l2 / den
    acc = None
    for q in range(4):
        xq = w1 * hq[(0, q)] + w2 * hq[(1, q)]
        t = jnp.dot(xq, w2q[q], preferred_element_type=_f32)
        acc = t if acc is None else acc + t
    yout[...] = acc


def _mid(svals, b1a, b1b, fw11, fb11, fw12, fb12, w2pad):
    sl = svals[:, :, :, N - 1:N, :]
    b1 = jnp.concatenate([b1a, b1b]).reshape(4, FQ)
    f11 = fw11[:, 0].reshape(4, FQ)
    f12 = fw12[:, 0].reshape(4, FQ)
    w2q = w2pad.reshape(4, FQ, CP)
    scspec = pl.BlockSpec((1, 1), lambda r: (0, 0))
    return pl.pallas_call(
        _mid_body,
        grid=(GRID,),
        in_specs=[
            pl.BlockSpec((2, 2, 2, BN, FQ), lambda r: (0, 0, 0, r, 0)),
            pl.BlockSpec((2, 2, 2, 1, FQ), lambda r: (0, 0, 0, 0, 0)),
            pl.BlockSpec((4, FQ), lambda r: (0, 0)),
            pl.BlockSpec((4, FQ), lambda r: (0, 0)),
            pl.BlockSpec((4, FQ), lambda r: (0, 0)),
            scspec, scspec,
            pl.BlockSpec((4, FQ, CP), lambda r: (0, 0, 0)),
        ],
        out_specs=pl.BlockSpec((BN, CP), lambda r: (r, 0)),
        out_shape=jax.ShapeDtypeStruct((N, CP), _f32),
    )(svals, sl, b1, f11, f12,
      fb11.reshape(1, 1), fb12.reshape(1, 1), w2q)


def _spmm48_body(y, srcs, dsts, ews, zrows, out,
                 idx_s, idx_d, ewt, sidx, didx, ewc, rows, acc, gsem):
    c = lax.axis_index("c")
    s = lax.axis_index("s")
    rbase = s * RPS
    pltpu.sync_copy(zrows.at[pl.ds(rbase, RPS)], acc.at[pl.ds(rbase, RPS)])
    pltpu.sync_copy(srcs.at[c, s], idx_s)
    pltpu.sync_copy(dsts.at[c, s], idx_d)
    pltpu.sync_copy(ews.at[c, s], ewt)
    plsc.subcore_barrier()

    @pl.loop(0, CH_PER_SUB)
    def _edges(j):
        for g in range(CHUNK // L):
            sidx[pl.ds(g * L, L)] = idx_s[j, pl.ds(g * L, L)]
            didx[pl.ds(g * L, L)] = idx_d[j, pl.ds(g * L, L)]
            ewc[pl.ds(g * L, L)] = ewt[j, pl.ds(g * L, L)]
        pltpu.async_copy(y.at[sidx], rows, gsem).wait()

        @pl.loop(0, CHUNK // L)
        def _grp(k):
            wg = ewc[pl.ds(k * L, L)]
            for u in range(L):
                e = k * L + u
                wv = _lane_splat(wg, u)
                for g in range(CP // L):
                    rows[e, pl.ds(g * L, L)] = rows[e, pl.ds(g * L, L)] * wv

        pltpu.sync_copy(rows, acc.at[didx], add=True)

    plsc.subcore_barrier()
    pltpu.sync_copy(acc.at[pl.ds(rbase, RPS)], out.at[c, pl.ds(rbase, RPS)])


def _spmm48(y, srcs, dsts, ews, zrows):
    mesh = plsc.VectorSubcoreMesh(core_axis_name="c", subcore_axis_name="s",
                                  num_cores=NC, num_subcores=NS)
    return pl.kernel(
        _spmm48_body,
        out_type=jax.ShapeDtypeStruct((NC, NP, CP), _f32),
        mesh=mesh,
        compiler_params=pltpu.CompilerParams(use_tc_tiling_on_sc=False),
        scratch_types=[
            pltpu.VMEM((CH_PER_SUB, CHUNK), jnp.int32),
            pltpu.VMEM((CH_PER_SUB, CHUNK), jnp.int32),
            pltpu.VMEM((CH_PER_SUB, CHUNK), _f32),
            pltpu.VMEM((CHUNK,), jnp.int32),
            pltpu.VMEM((CHUNK,), jnp.int32),
            pltpu.VMEM((CHUNK,), _f32),
            pltpu.VMEM((CHUNK, CP), _f32),
            pltpu.VMEM_SHARED((NP, CP), _f32),
            pltpu.SemaphoreType.DMA,
        ],
    )(y, srcs, dsts, ews, zrows)


def _fin_body(tvals, tl, b2p, f1, f2, fb1, fb2, out, gp1o, gp2o):
    gp1 = tvals[0] + b2p[...]
    gp2 = tvals[1] + b2p[...]
    tlv = tl[...]
    g1l = tlv[0] + b2p[...]
    g2l = tlv[1] + b2p[...]
    l1 = jax.nn.sigmoid(jnp.sum(g1l * f1[...]) + fb1[0, 0])
    l2 = jax.nn.sigmoid(jnp.sum(g2l * f2[...]) + fb2[0, 0])
    den = jnp.maximum(l1 + l2, 1e-12)
    out[...] = (l1 / den) * gp1 + (l2 / den) * gp2
    gp1o[...] = gp1
    gp2o[...] = gp2


def _fin(tvals, b2p, fw1p, fb1, fw2p, fb2):
    tl = tvals[:, N - 1, :]
    vspec = pl.BlockSpec((1, CP), lambda r: (0, 0))
    scspec = pl.BlockSpec((1, 1), lambda r: (0, 0))
    oshape = jax.ShapeDtypeStruct((N, CP), _f32)
    ospec = pl.BlockSpec((BN, CP), lambda r: (r, 0))
    return pl.pallas_call(
        _fin_body,
        grid=(GRID,),
        in_specs=[pl.BlockSpec((2, BN, CP), lambda r: (0, r, 0)),
                  pl.BlockSpec((2, CP), lambda r: (0, 0)),
                  vspec, vspec, vspec, scspec, scspec],
        out_specs=[ospec, ospec, ospec],
        out_shape=[oshape, oshape, oshape],
    )(tvals, tl, b2p.reshape(1, CP), fw1p.reshape(1, CP),
      fw2p.reshape(1, CP), fb1.reshape(1, 1), fb2.reshape(1, 1))


def _pad_edges(ei, ew):
    dst = ei[0].astype(jnp.int32)
    src = ei[1].astype(jnp.int32)
    pad = E_PAD - E
    dst = jnp.concatenate([dst, jnp.zeros((pad,), jnp.int32)])
    src = jnp.concatenate([src, jnp.zeros((pad,), jnp.int32)])
    ewp = jnp.concatenate([ew.astype(_f32), jnp.zeros((pad,), _f32)])
    shape = (NS, CH_PER_SUB, CHUNK)
    return dst.reshape(shape), src.reshape(shape), ewp.reshape(shape)


def kernel(x1_0, x1_1, x2_0, x2_1, edge_index1, edge_weight1,
           edge_index2, edge_weight2, W1a, b1a, W1b, b1b, W2, b2,
           fw11, fb11, fw12, fb12, fw1, fb1, fw2, fb2):
    d1, s1, w1 = _pad_edges(edge_index1, edge_weight1)
    d2, s2, w2 = _pad_edges(edge_index2, edge_weight2)
    srcs = jnp.stack([s1, s2])
    dsts = jnp.stack([d1, d2])
    ews = jnp.stack([w1, w2])

    z = _mm4(x1_0, x1_1, x2_0, x2_1, W1a, W1b)
    zflat = z.reshape(8 * N, FQ)

    zrows_h = jnp.zeros((NP, FQ), _f32)
    svals = _spmm256(zflat, srcs, dsts, ews, zrows_h)

    w2pad = jnp.concatenate([W2, jnp.zeros((2 * H, CP - C), _f32)], axis=1)
    y = _mid(svals, b1a, b1b, fw11, fb11, fw12, fb12, w2pad)

    zrows_c = jnp.zeros((NP, CP), _f32)
    tvals = _spmm48(y, srcs, dsts, ews, zrows_c)

    b2p = jnp.concatenate([b2, jnp.zeros((CP - C,), _f32)])
    fw1p = jnp.concatenate([fw1[:, 0], jnp.zeros((CP - C,), _f32)])
    fw2p = jnp.concatenate([fw2[:, 0], jnp.zeros((CP - C,), _f32)])
    out, gp1, gp2 = _fin(tvals, b2p, fw1p, fb1, fw2p, fb2)
    return (out[:, :C], gp1[:, :C], gp2[:, :C])
